# Initial kernel scaffold; baseline (speedup 1.0000x reference)
import jax
import jax.numpy as jnp
from jax.experimental import pallas as pl


def kernel(edge_index, edge_values, emb_table, W1_0, b1_0, W2_0, b2_0, W1_1, b1_1, W2_1, b2_1):
    raise NotImplementedError("write your pallas kernel here")


# speedup vs baseline: 3.6267x; 3.6267x over previous
import functools

import jax
import jax.numpy as jnp
from jax import lax
from jax.experimental import pallas as pl
from jax.experimental.pallas import tpu as pltpu
from jax.experimental.pallas import tpu_sc as plsc

N_NODES = 50000
NH = N_NODES // 2
ACC_ROWS = 25600
DUMMY = NH
EMB = 64
CHUNK = 128
EPT = 50048
E_PAD = 16 * EPT


def _spmm_body(row_hbm, col_hbm, val_hbm, feat_hbm, out_hbm,
               row_v, col_v, val_v, idx_v, rows_v, zbuf, acc, sem):
    c = lax.axis_index("c")
    s = lax.axis_index("s")
    base = c * NH

    def _zb(i, _):
        for j in range(4):
            zbuf[i, pl.ds(16 * j, 16)] = jnp.zeros((16,), jnp.float32)
        return 0
    lax.fori_loop(0, 200, _zb, 0)

    def _zacc(m, _):
        off = pl.multiple_of(s * 1600 + m * 200, 8)
        pltpu.sync_copy(zbuf, acc.at[pl.ds(off, 200)])
        return 0
    lax.fori_loop(0, 8, _zacc, 0)
    plsc.subcore_barrier()

    edge0 = s * EPT

    def _chunk(k, _):
        e0 = pl.multiple_of(edge0 + k * CHUNK, 8)
        pltpu.sync_copy(col_hbm.at[pl.ds(e0, CHUNK)], col_v)
        gather = pltpu.async_copy(feat_hbm.at[col_v], rows_v, sem)
        pltpu.sync_copy(row_hbm.at[pl.ds(e0, CHUNK)], row_v)
        pltpu.sync_copy(val_hbm.at[pl.ds(e0, CHUNK)], val_v)
        for i in range(CHUNK // 16):
            r16 = row_v[pl.ds(16 * i, 16)]
            owned = (r16 >= base) & (r16 < base + NH)
            idx_v[pl.ds(16 * i, 16)] = jnp.where(owned, r16 - base, DUMMY)
        gather.wait()

        def _scale(i, _):
            v16 = val_v[pl.ds(16 * i, 16)]
            for lane in range(16):
                r = 16 * i + lane
                v = v16[lane]
                for j in range(4):
                    rows_v[r, pl.ds(16 * j, 16)] = rows_v[r, pl.ds(16 * j, 16)] * v
            return 0
        lax.fori_loop(0, CHUNK // 16, _scale, 0)
        pltpu.sync_copy(rows_v, acc.at[idx_v], add=True)
        return 0
    lax.fori_loop(0, EPT // CHUNK, _chunk, 0)
    plsc.subcore_barrier()

    for m in range(8):
        ch = s + 16 * m

        @pl.when(ch < NH // 200)
        def _():
            src = pl.multiple_of(ch * 200, 8)
            dst = pl.multiple_of(base + ch * 200, 8)
            pltpu.sync_copy(acc.at[pl.ds(src, 200)], out_hbm.at[pl.ds(dst, 200)])


_spmm = functools.partial(
    pl.kernel,
    out_type=jax.ShapeDtypeStruct((N_NODES, EMB), jnp.float32),
    mesh=plsc.VectorSubcoreMesh(core_axis_name="c", subcore_axis_name="s"),
    compiler_params=pltpu.CompilerParams(use_tc_tiling_on_sc=False),
    scratch_types=[
        pltpu.VMEM((CHUNK,), jnp.int32),
        pltpu.VMEM((CHUNK,), jnp.int32),
        pltpu.VMEM((CHUNK,), jnp.float32),
        pltpu.VMEM((CHUNK,), jnp.int32),
        pltpu.VMEM((CHUNK, EMB), jnp.float32),
        pltpu.VMEM((200, EMB), jnp.float32),
        pltpu.VMEM_SHARED((ACC_ROWS, EMB), jnp.float32),
        pltpu.SemaphoreType.DMA,
    ],
)(_spmm_body)


def _dense(agg, feat, w1t, b1, w2t, b2):
    n = agg.shape[0]
    blk = 2000

    def body(a_ref, f_ref, w1_ref, b1_ref, w2_ref, b2_ref, nf_ref, nrm_ref):
        a = a_ref[...]
        f = f_ref[...]
        p1 = lax.dot(a, w1_ref[...], precision=lax.Precision.HIGHEST) + b1_ref[...]
        p1 = jnp.where(p1 >= 0, p1, 0.2 * p1)
        p2 = lax.dot(a * f, w2_ref[...], precision=lax.Precision.HIGHEST) + b2_ref[...]
        p2 = jnp.where(p2 >= 0, p2, 0.2 * p2)
        nf = p1 + p2
        nf_ref[...] = nf
        nrm = jnp.sqrt(jnp.sum(nf * nf, axis=1, keepdims=True))
        nrm_ref[...] = nf / jnp.maximum(nrm, 1e-12)

    return pl.pallas_call(
        body,
        grid=(n // blk,),
        in_specs=[
            pl.BlockSpec((blk, EMB), lambda i: (i, 0)),
            pl.BlockSpec((blk, EMB), lambda i: (i, 0)),
            pl.BlockSpec((EMB, EMB), lambda i: (0, 0)),
            pl.BlockSpec((1, EMB), lambda i: (0, 0)),
            pl.BlockSpec((EMB, EMB), lambda i: (0, 0)),
            pl.BlockSpec((1, EMB), lambda i: (0, 0)),
        ],
        out_specs=[
            pl.BlockSpec((blk, EMB), lambda i: (i, 0)),
            pl.BlockSpec((blk, EMB), lambda i: (i, 0)),
        ],
        out_shape=[
            jax.ShapeDtypeStruct((n, EMB), jnp.float32),
            jax.ShapeDtypeStruct((n, EMB), jnp.float32),
        ],
    )(agg, feat, w1t, b1.reshape(1, EMB), w2t, b2.reshape(1, EMB))


def kernel(edge_index, edge_values, emb_table,
           W1_0, b1_0, W2_0, b2_0, W1_1, b1_1, W2_1, b2_1):
    e = edge_values.shape[0]
    pad = E_PAD - e
    row = jnp.concatenate([edge_index[0], jnp.zeros((pad,), jnp.int32)])
    col = jnp.concatenate([edge_index[1], jnp.zeros((pad,), jnp.int32)])
    val = jnp.concatenate([edge_values, jnp.zeros((pad,), jnp.float32)])

    feat = emb_table
    outs = [emb_table]
    for (w1, b1, w2, b2) in ((W1_0, b1_0, W2_0, b2_0), (W1_1, b1_1, W2_1, b2_1)):
        agg = _spmm(row, col, val, feat)
        feat, nrm = _dense(agg, feat, w1.T, b1, w2.T, b2)
        outs.append(nrm)
    return tuple(outs)
